# Initial kernel scaffold; baseline (speedup 1.0000x reference)
import jax
import jax.numpy as jnp
from jax.experimental import pallas as pl


def kernel(raw_results):
    raise NotImplementedError("write your pallas kernel here")


# speedup vs baseline: 1.0135x; 1.0135x over previous
import functools

import jax
import jax.numpy as jnp
import numpy as np
from jax import lax
from jax.experimental import pallas as pl
from jax.experimental.pallas import tpu as pltpu
from jax.experimental.pallas import tpu_sc as plsc

MODES = 12
PHOTONS = 6
GENS = 4
BATCH = 1024
IMG = 100
PATCH = IMG * IMG // GENS

LANES = 16
NUM_CORES = 2
NUM_SUBCORES = 16
NUM_WORKERS = NUM_CORES * NUM_SUBCORES
ROWS_PER_WORKER = BATCH // NUM_WORKERS


def _enum_states(m, n):
    if m == 1:
        return [(n,)]
    out = []
    for k in range(n, -1, -1):
        for rest in _enum_states(m - 1, n - k):
            out.append((k,) + rest)
    return out


def _support_int(state):
    m = len(state)
    res = 0
    for i in range(m):
        if state[i] != 0:
            res += 2 ** (m - i)
    return res


def _build_tables():
    states = _enum_states(MODES, PHOTONS)
    k_total = len(states)
    sup = np.array([_support_int(s) for s in states], dtype=np.int64)
    uniq = np.sort(np.unique(sup))
    bin_of = {int(v): i for i, v in enumerate(uniq)}
    bin_idx = np.array([bin_of[int(v)] for v in sup], dtype=np.int32)
    n_bins = len(uniq)

    counts = np.bincount(bin_idx, minlength=n_bins)
    order = np.argsort(bin_idx, kind="stable").astype(np.int32)
    starts = np.zeros(n_bins + 1, dtype=np.int64)
    np.cumsum(counts, out=starts[1:])

    pad_slot = k_total
    trash_base = n_bins

    gtab = []
    btab = []
    wtab = []
    groups = []
    for ell in (1, 5, 10):
        bins_e = np.where(counts == ell)[0].astype(np.int32)
        n_vec = -(-len(bins_e) // LANES)
        groups.append((int(ell), int(n_vec), len(gtab) * LANES,
                       len(btab) * LANES))
        for v in range(n_vec):
            lane_bins = bins_e[v * LANES:(v + 1) * LANES]
            n_real = len(lane_bins)
            brow = np.full(LANES, 0, dtype=np.int32)
            wrow = np.zeros(LANES, dtype=np.float32)
            for ln in range(LANES):
                if ln < n_real:
                    c = int(lane_bins[ln])
                    brow[ln] = c
                    wrow[ln] = 1.0 if 4 <= c < 4 + PATCH else 0.0
                else:
                    brow[ln] = trash_base + 3 + ln
            btab.append(brow)
            wtab.append(wrow)
            for t in range(ell):
                grow = np.full(LANES, pad_slot, dtype=np.int32)
                for ln in range(n_real):
                    c = int(lane_bins[ln])
                    grow[ln] = int(order[starts[c] + t])
                gtab.append(grow)

    gtab = np.concatenate(gtab).astype(np.int32)
    btab = np.concatenate(btab).astype(np.int32)
    wtab = np.concatenate(wtab).astype(np.float32)
    return k_total, n_bins, gtab, btab, wtab, groups


K_TOTAL, N_BINS, _GTAB, _BTAB, _WTAB, _GROUPS = _build_tables()

ROW_BUF = K_TOTAL + 8
BIN_BUF = 2544
OUT_BUF = PATCH * GENS + 16
N_WVEC = -(-PATCH // LANES)


def _tec_body(raw_hbm, gtab_hbm, btab_hbm, wtab_hbm, out_hbm,
              row_v, bins_v, outrow_v, gtab_v, btab_v, wtab_v):
    wid = lax.axis_index("s") * NUM_CORES + lax.axis_index("c")
    zeros = jnp.zeros((LANES,), jnp.float32)
    neg_inf = jnp.full((LANES,), -jnp.inf, jnp.float32)
    lane = lax.iota(jnp.int32, LANES)

    pltpu.sync_copy(gtab_hbm, gtab_v)
    pltpu.sync_copy(btab_hbm, btab_v)
    pltpu.sync_copy(wtab_hbm, wtab_v)

    row_v[pl.ds(ROW_BUF - LANES, LANES)] = zeros
    for off in range(2496, BIN_BUF, LANES):
        bins_v[pl.ds(off, LANES)] = zeros

    def do_row(k, _):
        b = wid * ROWS_PER_WORKER + k
        for g in range(GENS):
            r = g * BATCH + b
            pltpu.sync_copy(raw_hbm.at[pl.ds(r * K_TOTAL, K_TOTAL)],
                            row_v.at[pl.ds(0, K_TOTAL)])

            carry = (zeros, neg_inf)

            def seg_vec(i, carry, ell=0, goff=0, boff=0):
                svec, mvec = carry
                acc = zeros
                for t in range(ell):
                    ix = gtab_v[pl.ds(goff + (i * ell + t) * LANES, LANES)]
                    acc = acc + plsc.load_gather(row_v, [ix])
                bix = btab_v[pl.ds(boff + i * LANES, LANES)]
                plsc.store_scatter(bins_v, [bix], acc)
                wv = wtab_v[pl.ds(boff + i * LANES, LANES)]
                mvec = jnp.maximum(mvec, jnp.where(wv > 0.5, acc, -jnp.inf))
                return svec + acc, mvec

            for ell, n_vec, goff, boff in _GROUPS:
                carry = lax.fori_loop(
                    0, n_vec,
                    functools.partial(seg_vec, ell=ell, goff=goff, boff=boff),
                    carry)
            svec, mvec = carry

            total = jnp.maximum(jnp.sum(svec), 1e-8)
            maxb = jnp.max(mvec)
            denom_v = jnp.full((LANES,), maxb + 1e-8 * total, jnp.float32)
            scale_v = 1.0 / denom_v

            gbase = g * PATCH

            def scale_vec(i, _):
                q = bins_v[pl.ds(4 + i * LANES, LANES)] * scale_v
                outrow_v[pl.ds(gbase + i * LANES, LANES)] = q
                return 0

            lax.fori_loop(0, N_WVEC - 1, scale_vec, 0)
            tail = (N_WVEC - 1) * LANES
            q = bins_v[pl.ds(4 + tail, LANES)] * scale_v
            q = jnp.where(lane < PATCH - tail, q, 0.0)
            outrow_v[pl.ds(gbase + tail, LANES)] = q

        pltpu.sync_copy(outrow_v.at[pl.ds(0, PATCH * GENS)],
                        out_hbm.at[pl.ds(b * (PATCH * GENS), PATCH * GENS)])
        return 0

    lax.fori_loop(0, ROWS_PER_WORKER, do_row, 0)


@jax.jit
def kernel(raw_results):
    raw1d = raw_results.reshape(GENS * BATCH * K_TOTAL)
    mesh = plsc.VectorSubcoreMesh(core_axis_name="c", subcore_axis_name="s")
    kfn = pl.kernel(
        _tec_body,
        out_type=jax.ShapeDtypeStruct((BATCH * PATCH * GENS,), jnp.float32),
        mesh=mesh,
        compiler_params=pltpu.CompilerParams(needs_layout_passes=False),
        scratch_types=[
            pltpu.VMEM((ROW_BUF,), jnp.float32),
            pltpu.VMEM((BIN_BUF,), jnp.float32),
            pltpu.VMEM((OUT_BUF,), jnp.float32),
            pltpu.VMEM((len(_GTAB),), jnp.int32),
            pltpu.VMEM((len(_BTAB),), jnp.int32),
            pltpu.VMEM((len(_WTAB),), jnp.float32),
        ],
    )
    out = kfn(raw1d, jnp.asarray(_GTAB), jnp.asarray(_BTAB),
              jnp.asarray(_WTAB))
    return out.reshape(BATCH, PATCH * GENS)


# speedup vs baseline: 1.0322x; 1.0185x over previous
import functools

import jax
import jax.numpy as jnp
import numpy as np
from jax import lax
from jax.experimental import pallas as pl
from jax.experimental.pallas import tpu as pltpu
from jax.experimental.pallas import tpu_sc as plsc

MODES = 12
PHOTONS = 6
GENS = 4
BATCH = 1024
IMG = 100
PATCH = IMG * IMG // GENS

LANES = 16
NUM_CORES = 2
NUM_SUBCORES = 16
NUM_WORKERS = NUM_CORES * NUM_SUBCORES
ROWS_PER_WORKER = BATCH // NUM_WORKERS


def _enum_states(m, n):
    if m == 1:
        return [(n,)]
    out = []
    for k in range(n, -1, -1):
        for rest in _enum_states(m - 1, n - k):
            out.append((k,) + rest)
    return out


def _support_int(state):
    m = len(state)
    res = 0
    for i in range(m):
        if state[i] != 0:
            res += 2 ** (m - i)
    return res


def _build_tables():
    states = _enum_states(MODES, PHOTONS)
    k_total = len(states)
    sup = np.array([_support_int(s) for s in states], dtype=np.int64)
    uniq = np.sort(np.unique(sup))
    bin_of = {int(v): i for i, v in enumerate(uniq)}
    bin_idx = np.array([bin_of[int(v)] for v in sup], dtype=np.int32)
    n_bins = len(uniq)

    counts = np.bincount(bin_idx, minlength=n_bins)
    order = np.argsort(bin_idx, kind="stable").astype(np.int32)
    starts = np.zeros(n_bins + 1, dtype=np.int64)
    np.cumsum(counts, out=starts[1:])

    pad_slot = k_total
    trash_base = n_bins

    gtab = []
    btab = []
    wtab = []
    groups = []
    for ell in (1, 5, 10):
        bins_e = np.where(counts == ell)[0].astype(np.int32)
        n_vec = -(-len(bins_e) // LANES)
        groups.append((int(ell), int(n_vec), len(gtab) * LANES,
                       len(btab) * LANES))
        for v in range(n_vec):
            lane_bins = bins_e[v * LANES:(v + 1) * LANES]
            n_real = len(lane_bins)
            brow = np.full(LANES, 0, dtype=np.int32)
            wrow = np.zeros(LANES, dtype=np.float32)
            for ln in range(LANES):
                if ln < n_real:
                    c = int(lane_bins[ln])
                    brow[ln] = c
                    wrow[ln] = 1.0 if 4 <= c < 4 + PATCH else 0.0
                else:
                    brow[ln] = trash_base + 3 + ln
            btab.append(brow)
            wtab.append(wrow)
            for t in range(ell):
                grow = np.full(LANES, pad_slot, dtype=np.int32)
                for ln in range(n_real):
                    c = int(lane_bins[ln])
                    grow[ln] = int(order[starts[c] + t])
                gtab.append(grow)

    gtab = np.concatenate(gtab).astype(np.int32)
    btab = np.concatenate(btab).astype(np.int32)
    wtab = np.concatenate(wtab).astype(np.float32)
    return k_total, n_bins, gtab, btab, wtab, groups


K_TOTAL, N_BINS, _GTAB, _BTAB, _WTAB, _GROUPS = _build_tables()

ROW_BUF = K_TOTAL + 8
BIN_BUF = 2544
OUT_BUF = PATCH * GENS + 16
N_WVEC = -(-PATCH // LANES)


def _tec_body(raw_hbm, gtab_hbm, btab_hbm, wtab_hbm, out_hbm,
              row_v, bins_v, outrow_v, gtab_v, btab_v, wtab_v):
    wid = lax.axis_index("s") * NUM_CORES + lax.axis_index("c")
    zeros = jnp.zeros((LANES,), jnp.float32)
    neg_inf = jnp.full((LANES,), -jnp.inf, jnp.float32)
    lane = lax.iota(jnp.int32, LANES)

    pltpu.sync_copy(gtab_hbm, gtab_v)
    pltpu.sync_copy(btab_hbm, btab_v)
    pltpu.sync_copy(wtab_hbm, wtab_v)

    row_v[pl.ds(ROW_BUF - LANES, LANES)] = zeros
    for off in range(2496, BIN_BUF, LANES):
        bins_v[pl.ds(off, LANES)] = zeros

    def do_row(k, _):
        b = wid * ROWS_PER_WORKER + k
        for g in range(GENS):
            r = g * BATCH + b
            pltpu.sync_copy(raw_hbm.at[pl.ds(r * K_TOTAL, K_TOTAL)],
                            row_v.at[pl.ds(0, K_TOTAL)])

            carry = (zeros, neg_inf)

            def seg_vec(i, carry, ell=0, goff=0, boff=0):
                svec, mvec = carry
                vals = []
                for t in range(ell):
                    ix = gtab_v[pl.ds(goff + (i * ell + t) * LANES, LANES)]
                    vals.append(plsc.load_gather(row_v, [ix]))
                while len(vals) > 1:
                    nxt = [vals[j] + vals[j + 1]
                           for j in range(0, len(vals) - 1, 2)]
                    if len(vals) % 2:
                        nxt.append(vals[-1])
                    vals = nxt
                acc = vals[0]
                bix = btab_v[pl.ds(boff + i * LANES, LANES)]
                plsc.store_scatter(bins_v, [bix], acc)
                wv = wtab_v[pl.ds(boff + i * LANES, LANES)]
                mvec = jnp.maximum(mvec, jnp.where(wv > 0.5, acc, -jnp.inf))
                return svec + acc, mvec

            for ell, n_vec, goff, boff in _GROUPS:
                carry = lax.fori_loop(
                    0, n_vec,
                    functools.partial(seg_vec, ell=ell, goff=goff, boff=boff),
                    carry)
            svec, mvec = carry

            total = jnp.maximum(jnp.sum(svec), 1e-8)
            maxb = jnp.max(mvec)
            denom_v = jnp.full((LANES,), maxb + 1e-8 * total, jnp.float32)
            scale_v = 1.0 / denom_v

            gbase = g * PATCH

            def scale_vec(i, _):
                q = bins_v[pl.ds(4 + i * LANES, LANES)] * scale_v
                outrow_v[pl.ds(gbase + i * LANES, LANES)] = q
                return 0

            lax.fori_loop(0, N_WVEC - 1, scale_vec, 0)
            tail = (N_WVEC - 1) * LANES
            q = bins_v[pl.ds(4 + tail, LANES)] * scale_v
            q = jnp.where(lane < PATCH - tail, q, 0.0)
            outrow_v[pl.ds(gbase + tail, LANES)] = q

        pltpu.sync_copy(outrow_v.at[pl.ds(0, PATCH * GENS)],
                        out_hbm.at[pl.ds(b * (PATCH * GENS), PATCH * GENS)])
        return 0

    lax.fori_loop(0, ROWS_PER_WORKER, do_row, 0)


@jax.jit
def kernel(raw_results):
    raw1d = raw_results.reshape(GENS * BATCH * K_TOTAL)
    mesh = plsc.VectorSubcoreMesh(core_axis_name="c", subcore_axis_name="s")
    kfn = pl.kernel(
        _tec_body,
        out_type=jax.ShapeDtypeStruct((BATCH * PATCH * GENS,), jnp.float32),
        mesh=mesh,
        compiler_params=pltpu.CompilerParams(needs_layout_passes=False),
        scratch_types=[
            pltpu.VMEM((ROW_BUF,), jnp.float32),
            pltpu.VMEM((BIN_BUF,), jnp.float32),
            pltpu.VMEM((OUT_BUF,), jnp.float32),
            pltpu.VMEM((len(_GTAB),), jnp.int32),
            pltpu.VMEM((len(_BTAB),), jnp.int32),
            pltpu.VMEM((len(_WTAB),), jnp.float32),
        ],
    )
    out = kfn(raw1d, jnp.asarray(_GTAB), jnp.asarray(_BTAB),
              jnp.asarray(_WTAB))
    return out.reshape(BATCH, PATCH * GENS)


# speedup vs baseline: 1.1982x; 1.1609x over previous
import functools

import jax
import jax.numpy as jnp
import numpy as np
from jax import lax
from jax.experimental import pallas as pl
from jax.experimental.pallas import tpu as pltpu
from jax.experimental.pallas import tpu_sc as plsc

MODES = 12
PHOTONS = 6
GENS = 4
BATCH = 1024
IMG = 100
PATCH = IMG * IMG // GENS

LANES = 16
NUM_CORES = 2
NUM_SUBCORES = 16
NUM_WORKERS = NUM_CORES * NUM_SUBCORES
ROWS_PER_WORKER = BATCH // NUM_WORKERS


def _enum_states(m, n):
    if m == 1:
        return [(n,)]
    out = []
    for k in range(n, -1, -1):
        for rest in _enum_states(m - 1, n - k):
            out.append((k,) + rest)
    return out


def _support_int(state):
    m = len(state)
    res = 0
    for i in range(m):
        if state[i] != 0:
            res += 2 ** (m - i)
    return res


def _build_tables():
    states = _enum_states(MODES, PHOTONS)
    k_total = len(states)
    sup = np.array([_support_int(s) for s in states], dtype=np.int64)
    uniq = np.sort(np.unique(sup))
    bin_of = {int(v): i for i, v in enumerate(uniq)}
    bin_idx = np.array([bin_of[int(v)] for v in sup], dtype=np.int32)
    n_bins = len(uniq)

    counts = np.bincount(bin_idx, minlength=n_bins)
    order = np.argsort(bin_idx, kind="stable").astype(np.int32)
    starts = np.zeros(n_bins + 1, dtype=np.int64)
    np.cumsum(counts, out=starts[1:])

    pad_slot = k_total
    trash_base = n_bins

    gtab = []
    btab = []
    wtab = []
    groups = []
    for ell in (1, 5, 10):
        bins_e = np.where(counts == ell)[0].astype(np.int32)
        n_vec = -(-len(bins_e) // LANES)
        groups.append((int(ell), int(n_vec), len(gtab) * LANES,
                       len(btab) * LANES))
        for v in range(n_vec):
            lane_bins = bins_e[v * LANES:(v + 1) * LANES]
            n_real = len(lane_bins)
            brow = np.full(LANES, 0, dtype=np.int32)
            wrow = np.zeros(LANES, dtype=np.float32)
            for ln in range(LANES):
                if ln < n_real:
                    c = int(lane_bins[ln])
                    brow[ln] = c
                    wrow[ln] = 1.0 if 4 <= c < 4 + PATCH else 0.0
                else:
                    brow[ln] = trash_base + 3 + ln
            btab.append(brow)
            wtab.append(wrow)
            for t in range(ell):
                grow = np.full(LANES, pad_slot, dtype=np.int32)
                for ln in range(n_real):
                    c = int(lane_bins[ln])
                    grow[ln] = int(order[starts[c] + t])
                gtab.append(grow)

    gtab = np.concatenate(gtab).astype(np.int32)
    btab = np.concatenate(btab).astype(np.int32)
    wtab = np.concatenate(wtab).astype(np.float32)
    return k_total, n_bins, gtab, btab, wtab, groups


K_TOTAL, N_BINS, _GTAB, _BTAB, _WTAB, _GROUPS = _build_tables()

ROW_BUF = K_TOTAL + 8
BIN_BUF = 2544
OUT_BUF = PATCH * GENS + 16
N_WVEC = -(-PATCH // LANES)


def _tec_body(raw_hbm, gtab_hbm, btab_hbm, wtab_hbm, out_hbm,
              row_v0, row_v1, bins_v, outrow_v, gtab_v, btab_v, wtab_v, sem):
    wid = lax.axis_index("s") * NUM_CORES + lax.axis_index("c")
    zeros = jnp.zeros((LANES,), jnp.float32)
    neg_inf = jnp.full((LANES,), -jnp.inf, jnp.float32)
    lane = lax.iota(jnp.int32, LANES)

    pltpu.sync_copy(gtab_hbm, gtab_v)
    pltpu.sync_copy(btab_hbm, btab_v)
    pltpu.sync_copy(wtab_hbm, wtab_v)

    row_v0[pl.ds(ROW_BUF - LANES, LANES)] = zeros
    row_v1[pl.ds(ROW_BUF - LANES, LANES)] = zeros
    for off in range(2496, BIN_BUF, LANES):
        bins_v[pl.ds(off, LANES)] = zeros

    bufs = (row_v0, row_v1)
    b0 = wid * ROWS_PER_WORKER

    pltpu.async_copy(raw_hbm.at[pl.ds(b0 * K_TOTAL, K_TOTAL)],
                     row_v0.at[pl.ds(0, K_TOTAL)], sem)

    def do_row(k, _):
        b = wid * ROWS_PER_WORKER + k
        for g in range(GENS):
            row_v = bufs[g % 2]
            nr = (g + 1) * BATCH + b if g < GENS - 1 else b + 1
            pltpu.async_copy(raw_hbm.at[pl.ds(nr * K_TOTAL, K_TOTAL)],
                             bufs[(g + 1) % 2].at[pl.ds(0, K_TOTAL)], sem)
            pltpu.make_async_copy(raw_hbm.at[pl.ds(0, K_TOTAL)],
                                  row_v.at[pl.ds(0, K_TOTAL)], sem).wait()

            carry = (zeros, neg_inf)

            def seg_vec(i, carry, ell=0, goff=0, boff=0):
                svec, mvec = carry
                vals = []
                for t in range(ell):
                    ix = gtab_v[pl.ds(goff + (i * ell + t) * LANES, LANES)]
                    vals.append(plsc.load_gather(row_v, [ix]))
                while len(vals) > 1:
                    nxt = [vals[j] + vals[j + 1]
                           for j in range(0, len(vals) - 1, 2)]
                    if len(vals) % 2:
                        nxt.append(vals[-1])
                    vals = nxt
                acc = vals[0]
                bix = btab_v[pl.ds(boff + i * LANES, LANES)]
                plsc.store_scatter(bins_v, [bix], acc)
                wv = wtab_v[pl.ds(boff + i * LANES, LANES)]
                mvec = jnp.maximum(mvec, jnp.where(wv > 0.5, acc, -jnp.inf))
                return svec + acc, mvec

            for ell, n_vec, goff, boff in _GROUPS:
                carry = lax.fori_loop(
                    0, n_vec,
                    functools.partial(seg_vec, ell=ell, goff=goff, boff=boff),
                    carry)
            svec, mvec = carry

            total = jnp.maximum(jnp.sum(svec), 1e-8)
            maxb = jnp.max(mvec)
            denom_v = jnp.full((LANES,), maxb + 1e-8 * total, jnp.float32)
            scale_v = 1.0 / denom_v

            gbase = g * PATCH

            def scale_vec(i, _):
                q = bins_v[pl.ds(4 + i * LANES, LANES)] * scale_v
                outrow_v[pl.ds(gbase + i * LANES, LANES)] = q
                return 0

            lax.fori_loop(0, N_WVEC - 1, scale_vec, 0)
            tail = (N_WVEC - 1) * LANES
            q = bins_v[pl.ds(4 + tail, LANES)] * scale_v
            q = jnp.where(lane < PATCH - tail, q, 0.0)
            outrow_v[pl.ds(gbase + tail, LANES)] = q

        pltpu.sync_copy(outrow_v.at[pl.ds(0, PATCH * GENS)],
                        out_hbm.at[pl.ds(b * (PATCH * GENS), PATCH * GENS)])
        return 0

    lax.fori_loop(0, ROWS_PER_WORKER, do_row, 0)
    pltpu.make_async_copy(raw_hbm.at[pl.ds(0, K_TOTAL)],
                          row_v0.at[pl.ds(0, K_TOTAL)], sem).wait()


@jax.jit
def kernel(raw_results):
    raw1d = raw_results.reshape(GENS * BATCH * K_TOTAL)
    mesh = plsc.VectorSubcoreMesh(core_axis_name="c", subcore_axis_name="s")
    kfn = pl.kernel(
        _tec_body,
        out_type=jax.ShapeDtypeStruct((BATCH * PATCH * GENS,), jnp.float32),
        mesh=mesh,
        compiler_params=pltpu.CompilerParams(needs_layout_passes=False),
        scratch_types=[
            pltpu.VMEM((ROW_BUF,), jnp.float32),
            pltpu.VMEM((ROW_BUF,), jnp.float32),
            pltpu.VMEM((BIN_BUF,), jnp.float32),
            pltpu.VMEM((OUT_BUF,), jnp.float32),
            pltpu.VMEM((len(_GTAB),), jnp.int32),
            pltpu.VMEM((len(_BTAB),), jnp.int32),
            pltpu.VMEM((len(_WTAB),), jnp.float32),
            pltpu.SemaphoreType.DMA,
        ],
    )
    out = kfn(raw1d, jnp.asarray(_GTAB), jnp.asarray(_BTAB),
              jnp.asarray(_WTAB))
    return out.reshape(BATCH, PATCH * GENS)


# speedup vs baseline: 1.2119x; 1.0114x over previous
import functools

import jax
import jax.numpy as jnp
import numpy as np
from jax import lax
from jax.experimental import pallas as pl
from jax.experimental.pallas import tpu as pltpu
from jax.experimental.pallas import tpu_sc as plsc

MODES = 12
PHOTONS = 6
GENS = 4
BATCH = 1024
IMG = 100
PATCH = IMG * IMG // GENS

LANES = 16
NUM_CORES = 2
NUM_SUBCORES = 16
NUM_WORKERS = NUM_CORES * NUM_SUBCORES
ROWS_PER_WORKER = BATCH // NUM_WORKERS


def _enum_states(m, n):
    if m == 1:
        return [(n,)]
    out = []
    for k in range(n, -1, -1):
        for rest in _enum_states(m - 1, n - k):
            out.append((k,) + rest)
    return out


def _support_int(state):
    m = len(state)
    res = 0
    for i in range(m):
        if state[i] != 0:
            res += 2 ** (m - i)
    return res


def _build_tables():
    states = _enum_states(MODES, PHOTONS)
    k_total = len(states)
    sup = np.array([_support_int(s) for s in states], dtype=np.int64)
    uniq = np.sort(np.unique(sup))
    bin_of = {int(v): i for i, v in enumerate(uniq)}
    bin_idx = np.array([bin_of[int(v)] for v in sup], dtype=np.int32)
    n_bins = len(uniq)

    counts = np.bincount(bin_idx, minlength=n_bins)
    order = np.argsort(bin_idx, kind="stable").astype(np.int32)
    starts = np.zeros(n_bins + 1, dtype=np.int64)
    np.cumsum(counts, out=starts[1:])

    pad_slot = k_total
    trash_base = n_bins

    gtab = []
    btab = []
    wtab = []
    groups = []
    for ell in (1, 5, 10):
        bins_e = np.where(counts == ell)[0].astype(np.int32)
        n_vec = -(-len(bins_e) // LANES)
        groups.append((int(ell), int(n_vec), len(gtab) * LANES,
                       len(btab) * LANES))
        for v in range(n_vec):
            lane_bins = bins_e[v * LANES:(v + 1) * LANES]
            n_real = len(lane_bins)
            brow = np.full(LANES, 0, dtype=np.int32)
            wrow = np.zeros(LANES, dtype=np.float32)
            for ln in range(LANES):
                if ln < n_real:
                    c = int(lane_bins[ln])
                    brow[ln] = c
                    wrow[ln] = 1.0 if 4 <= c < 4 + PATCH else 0.0
                else:
                    brow[ln] = trash_base + 3 + ln
            btab.append(brow)
            wtab.append(wrow)
            for t in range(ell):
                grow = np.full(LANES, pad_slot, dtype=np.int32)
                for ln in range(n_real):
                    c = int(lane_bins[ln])
                    grow[ln] = int(order[starts[c] + t])
                gtab.append(grow)

    gtab = np.concatenate(gtab).astype(np.int32)
    btab = np.concatenate(btab).astype(np.int32)
    wtab = np.concatenate(wtab).astype(np.float32)
    return k_total, n_bins, gtab, btab, wtab, groups


K_TOTAL, N_BINS, _GTAB, _BTAB, _WTAB, _GROUPS = _build_tables()

ROW_BUF = K_TOTAL + 8
BIN_BUF = 2544
OUT_BUF = PATCH * GENS + 16
N_WVEC = -(-PATCH // LANES)


def _tec_body(raw_hbm, gtab_hbm, btab_hbm, wtab_hbm, out_hbm,
              row_v0, row_v1, bins_v, outrow_v, gtab_v, btab_v, wtab_v,
              sem, sem_out):
    wid = lax.axis_index("s") * NUM_CORES + lax.axis_index("c")
    zeros = jnp.zeros((LANES,), jnp.float32)
    neg_inf = jnp.full((LANES,), -jnp.inf, jnp.float32)
    lane = lax.iota(jnp.int32, LANES)

    pltpu.sync_copy(gtab_hbm, gtab_v)
    pltpu.sync_copy(btab_hbm, btab_v)
    pltpu.sync_copy(wtab_hbm, wtab_v)

    row_v0[pl.ds(ROW_BUF - LANES, LANES)] = zeros
    row_v1[pl.ds(ROW_BUF - LANES, LANES)] = zeros
    for off in range(2496, BIN_BUF, LANES):
        bins_v[pl.ds(off, LANES)] = zeros

    bufs = (row_v0, row_v1)
    b0 = wid * ROWS_PER_WORKER

    pltpu.async_copy(raw_hbm.at[pl.ds(b0 * K_TOTAL, K_TOTAL)],
                     row_v0.at[pl.ds(0, K_TOTAL)], sem)
    pltpu.async_copy(outrow_v.at[pl.ds(0, PATCH * GENS)],
                     out_hbm.at[pl.ds(b0 * (PATCH * GENS), PATCH * GENS)],
                     sem_out)

    def do_row(k, _):
        b = wid * ROWS_PER_WORKER + k
        for g in range(GENS):
            row_v = bufs[g % 2]
            nr = (g + 1) * BATCH + b if g < GENS - 1 else b + 1
            pltpu.async_copy(raw_hbm.at[pl.ds(nr * K_TOTAL, K_TOTAL)],
                             bufs[(g + 1) % 2].at[pl.ds(0, K_TOTAL)], sem)
            pltpu.make_async_copy(raw_hbm.at[pl.ds(0, K_TOTAL)],
                                  row_v.at[pl.ds(0, K_TOTAL)], sem).wait()

            carry = (zeros, neg_inf)

            def seg_vec(i, carry, ell=0, goff=0, boff=0):
                svec, mvec = carry
                vals = []
                for t in range(ell):
                    ix = gtab_v[pl.ds(goff + (i * ell + t) * LANES, LANES)]
                    vals.append(plsc.load_gather(row_v, [ix]))
                while len(vals) > 1:
                    nxt = [vals[j] + vals[j + 1]
                           for j in range(0, len(vals) - 1, 2)]
                    if len(vals) % 2:
                        nxt.append(vals[-1])
                    vals = nxt
                acc = vals[0]
                bix = btab_v[pl.ds(boff + i * LANES, LANES)]
                plsc.store_scatter(bins_v, [bix], acc)
                wv = wtab_v[pl.ds(boff + i * LANES, LANES)]
                mvec = jnp.maximum(mvec, jnp.where(wv > 0.5, acc, -jnp.inf))
                return svec + acc, mvec

            for ell, n_vec, goff, boff in _GROUPS:
                carry = lax.fori_loop(
                    0, n_vec,
                    functools.partial(seg_vec, ell=ell, goff=goff, boff=boff),
                    carry)
            svec, mvec = carry

            if g == 0:
                pltpu.make_async_copy(
                    outrow_v.at[pl.ds(0, PATCH * GENS)],
                    out_hbm.at[pl.ds(b * (PATCH * GENS), PATCH * GENS)],
                    sem_out).wait()

            total = jnp.maximum(jnp.sum(svec), 1e-8)
            maxb = jnp.max(mvec)
            denom_v = jnp.full((LANES,), maxb + 1e-8 * total, jnp.float32)
            scale_v = 1.0 / denom_v

            gbase = g * PATCH

            def scale_vec(i, _):
                q = bins_v[pl.ds(4 + i * LANES, LANES)] * scale_v
                outrow_v[pl.ds(gbase + i * LANES, LANES)] = q
                return 0

            lax.fori_loop(0, N_WVEC - 1, scale_vec, 0)
            tail = (N_WVEC - 1) * LANES
            q = bins_v[pl.ds(4 + tail, LANES)] * scale_v
            q = jnp.where(lane < PATCH - tail, q, 0.0)
            outrow_v[pl.ds(gbase + tail, LANES)] = q

        pltpu.async_copy(outrow_v.at[pl.ds(0, PATCH * GENS)],
                         out_hbm.at[pl.ds(b * (PATCH * GENS), PATCH * GENS)],
                         sem_out)
        return 0

    lax.fori_loop(0, ROWS_PER_WORKER, do_row, 0)
    pltpu.make_async_copy(raw_hbm.at[pl.ds(0, K_TOTAL)],
                          row_v0.at[pl.ds(0, K_TOTAL)], sem).wait()
    pltpu.make_async_copy(outrow_v.at[pl.ds(0, PATCH * GENS)],
                          out_hbm.at[pl.ds(b0 * (PATCH * GENS), PATCH * GENS)],
                          sem_out).wait()


@jax.jit
def kernel(raw_results):
    raw1d = raw_results.reshape(GENS * BATCH * K_TOTAL)
    mesh = plsc.VectorSubcoreMesh(core_axis_name="c", subcore_axis_name="s")
    kfn = pl.kernel(
        _tec_body,
        out_type=jax.ShapeDtypeStruct((BATCH * PATCH * GENS,), jnp.float32),
        mesh=mesh,
        compiler_params=pltpu.CompilerParams(needs_layout_passes=False),
        scratch_types=[
            pltpu.VMEM((ROW_BUF,), jnp.float32),
            pltpu.VMEM((ROW_BUF,), jnp.float32),
            pltpu.VMEM((BIN_BUF,), jnp.float32),
            pltpu.VMEM((OUT_BUF,), jnp.float32),
            pltpu.VMEM((len(_GTAB),), jnp.int32),
            pltpu.VMEM((len(_BTAB),), jnp.int32),
            pltpu.VMEM((len(_WTAB),), jnp.float32),
            pltpu.SemaphoreType.DMA,
            pltpu.SemaphoreType.DMA,
        ],
    )
    out = kfn(raw1d, jnp.asarray(_GTAB), jnp.asarray(_BTAB),
              jnp.asarray(_WTAB))
    return out.reshape(BATCH, PATCH * GENS)


# speedup vs baseline: 1.2122x; 1.0003x over previous
import functools

import jax
import jax.numpy as jnp
import numpy as np
from jax import lax
from jax.experimental import pallas as pl
from jax.experimental.pallas import tpu as pltpu
from jax.experimental.pallas import tpu_sc as plsc

MODES = 12
PHOTONS = 6
GENS = 4
BATCH = 1024
IMG = 100
PATCH = IMG * IMG // GENS

LANES = 16
NUM_CORES = 2
NUM_SUBCORES = 16
NUM_WORKERS = NUM_CORES * NUM_SUBCORES
ROWS_PER_WORKER = BATCH // NUM_WORKERS


def _enum_states(m, n):
    if m == 1:
        return [(n,)]
    out = []
    for k in range(n, -1, -1):
        for rest in _enum_states(m - 1, n - k):
            out.append((k,) + rest)
    return out


def _support_int(state):
    m = len(state)
    res = 0
    for i in range(m):
        if state[i] != 0:
            res += 2 ** (m - i)
    return res


def _build_tables():
    states = _enum_states(MODES, PHOTONS)
    k_total = len(states)
    sup = np.array([_support_int(s) for s in states], dtype=np.int64)
    uniq = np.sort(np.unique(sup))
    bin_of = {int(v): i for i, v in enumerate(uniq)}
    bin_idx = np.array([bin_of[int(v)] for v in sup], dtype=np.int32)
    n_bins = len(uniq)

    counts = np.bincount(bin_idx, minlength=n_bins)
    order = np.argsort(bin_idx, kind="stable").astype(np.int32)
    starts = np.zeros(n_bins + 1, dtype=np.int64)
    np.cumsum(counts, out=starts[1:])

    pad_slot = k_total
    trash_base = n_bins

    gtab = []
    btab = []
    wtab = []
    groups = []
    for ell in (1, 5, 10):
        bins_e = np.where(counts == ell)[0].astype(np.int32)
        n_vec = -(-len(bins_e) // LANES)
        groups.append((int(ell), int(n_vec), len(gtab) * LANES,
                       len(btab) * LANES))
        for v in range(n_vec):
            lane_bins = bins_e[v * LANES:(v + 1) * LANES]
            n_real = len(lane_bins)
            brow = np.full(LANES, 0, dtype=np.int32)
            wrow = np.zeros(LANES, dtype=np.float32)
            for ln in range(LANES):
                if ln < n_real:
                    c = int(lane_bins[ln])
                    brow[ln] = c
                    wrow[ln] = 1.0 if 4 <= c < 4 + PATCH else 0.0
                else:
                    brow[ln] = trash_base + 3 + ln
            btab.append(brow)
            wtab.append(wrow)
            for t in range(ell):
                grow = np.full(LANES, pad_slot, dtype=np.int32)
                for ln in range(n_real):
                    c = int(lane_bins[ln])
                    grow[ln] = int(order[starts[c] + t])
                gtab.append(grow)

    gtab = np.concatenate(gtab).astype(np.int32)
    btab = np.concatenate(btab).astype(np.int32)
    wtab = np.concatenate(wtab).astype(np.float32)
    return k_total, n_bins, gtab, btab, wtab, groups


K_TOTAL, N_BINS, _GTAB, _BTAB, _WTAB, _GROUPS = _build_tables()

ROW_BUF = K_TOTAL + 8
BIN_BUF = 2544
OUT_BUF = PATCH * GENS + 16
N_WVEC = -(-PATCH // LANES)


def _tec_body(raw_hbm, gtab_hbm, btab_hbm, wtab_hbm, out_hbm,
              row_v0, row_v1, row_v2, row_v3, bins_v, outrow_v,
              gtab_v, btab_v, wtab_v, sem, sem_out):
    wid = lax.axis_index("s") * NUM_CORES + lax.axis_index("c")
    zeros = jnp.zeros((LANES,), jnp.float32)
    neg_inf = jnp.full((LANES,), -jnp.inf, jnp.float32)
    lane = lax.iota(jnp.int32, LANES)

    pltpu.sync_copy(gtab_hbm, gtab_v)
    pltpu.sync_copy(btab_hbm, btab_v)
    pltpu.sync_copy(wtab_hbm, wtab_v)

    bufs = (row_v0, row_v1, row_v2, row_v3)
    for rv in bufs:
        rv[pl.ds(ROW_BUF - LANES, LANES)] = zeros
    for off in range(2496, BIN_BUF, LANES):
        bins_v[pl.ds(off, LANES)] = zeros

    b0 = wid * ROWS_PER_WORKER

    for g in range(3):
        pltpu.async_copy(
            raw_hbm.at[pl.ds((g * BATCH + b0) * K_TOTAL, K_TOTAL)],
            bufs[g].at[pl.ds(0, K_TOTAL)], sem)
    pltpu.async_copy(outrow_v.at[pl.ds(0, PATCH * GENS)],
                     out_hbm.at[pl.ds(b0 * (PATCH * GENS), PATCH * GENS)],
                     sem_out)

    def do_row(k, _):
        b = wid * ROWS_PER_WORKER + k
        for g in range(GENS):
            row_v = bufs[g]
            gn = (g + 3) % GENS
            nr = gn * BATCH + b + (g + 3) // GENS
            pltpu.async_copy(raw_hbm.at[pl.ds(nr * K_TOTAL, K_TOTAL)],
                             bufs[gn].at[pl.ds(0, K_TOTAL)], sem)
            pltpu.make_async_copy(raw_hbm.at[pl.ds(0, K_TOTAL)],
                                  row_v.at[pl.ds(0, K_TOTAL)], sem).wait()

            carry = (zeros, neg_inf)

            def seg_vec(i, carry, ell=0, goff=0, boff=0):
                svec, mvec = carry
                vals = []
                for t in range(ell):
                    ix = gtab_v[pl.ds(goff + (i * ell + t) * LANES, LANES)]
                    vals.append(plsc.load_gather(row_v, [ix]))
                while len(vals) > 1:
                    nxt = [vals[j] + vals[j + 1]
                           for j in range(0, len(vals) - 1, 2)]
                    if len(vals) % 2:
                        nxt.append(vals[-1])
                    vals = nxt
                acc = vals[0]
                bix = btab_v[pl.ds(boff + i * LANES, LANES)]
                plsc.store_scatter(bins_v, [bix], acc)
                wv = wtab_v[pl.ds(boff + i * LANES, LANES)]
                mvec = jnp.maximum(mvec, jnp.where(wv > 0.5, acc, -jnp.inf))
                return svec + acc, mvec

            for ell, n_vec, goff, boff in _GROUPS:
                carry = lax.fori_loop(
                    0, n_vec,
                    functools.partial(seg_vec, ell=ell, goff=goff, boff=boff),
                    carry)
            svec, mvec = carry

            if g == 0:
                pltpu.make_async_copy(
                    outrow_v.at[pl.ds(0, PATCH * GENS)],
                    out_hbm.at[pl.ds(b * (PATCH * GENS), PATCH * GENS)],
                    sem_out).wait()

            total = jnp.maximum(jnp.sum(svec), 1e-8)
            maxb = jnp.max(mvec)
            denom_v = jnp.full((LANES,), maxb + 1e-8 * total, jnp.float32)
            scale_v = 1.0 / denom_v

            gbase = g * PATCH

            def scale_vec(i, _):
                q = bins_v[pl.ds(4 + i * LANES, LANES)] * scale_v
                outrow_v[pl.ds(gbase + i * LANES, LANES)] = q
                return 0

            lax.fori_loop(0, N_WVEC - 1, scale_vec, 0)
            tail = (N_WVEC - 1) * LANES
            q = bins_v[pl.ds(4 + tail, LANES)] * scale_v
            q = jnp.where(lane < PATCH - tail, q, 0.0)
            outrow_v[pl.ds(gbase + tail, LANES)] = q

        pltpu.async_copy(outrow_v.at[pl.ds(0, PATCH * GENS)],
                         out_hbm.at[pl.ds(b * (PATCH * GENS), PATCH * GENS)],
                         sem_out)
        return 0

    lax.fori_loop(0, ROWS_PER_WORKER, do_row, 0)
    for g in range(3):
        pltpu.make_async_copy(raw_hbm.at[pl.ds(0, K_TOTAL)],
                              bufs[g].at[pl.ds(0, K_TOTAL)], sem).wait()
    pltpu.make_async_copy(outrow_v.at[pl.ds(0, PATCH * GENS)],
                          out_hbm.at[pl.ds(b0 * (PATCH * GENS), PATCH * GENS)],
                          sem_out).wait()


@jax.jit
def kernel(raw_results):
    raw1d = raw_results.reshape(GENS * BATCH * K_TOTAL)
    mesh = plsc.VectorSubcoreMesh(core_axis_name="c", subcore_axis_name="s")
    kfn = pl.kernel(
        _tec_body,
        out_type=jax.ShapeDtypeStruct((BATCH * PATCH * GENS,), jnp.float32),
        mesh=mesh,
        compiler_params=pltpu.CompilerParams(needs_layout_passes=False),
        scratch_types=[
            pltpu.VMEM((ROW_BUF,), jnp.float32),
            pltpu.VMEM((ROW_BUF,), jnp.float32),
            pltpu.VMEM((ROW_BUF,), jnp.float32),
            pltpu.VMEM((ROW_BUF,), jnp.float32),
            pltpu.VMEM((BIN_BUF,), jnp.float32),
            pltpu.VMEM((OUT_BUF,), jnp.float32),
            pltpu.VMEM((len(_GTAB),), jnp.int32),
            pltpu.VMEM((len(_BTAB),), jnp.int32),
            pltpu.VMEM((len(_WTAB),), jnp.float32),
            pltpu.SemaphoreType.DMA,
            pltpu.SemaphoreType.DMA,
        ],
    )
    out = kfn(raw1d, jnp.asarray(_GTAB), jnp.asarray(_BTAB),
              jnp.asarray(_WTAB))
    return out.reshape(BATCH, PATCH * GENS)
